# baseline (speedup 1.0000x reference)
import dataclasses
import functools

import jax
import jax.numpy as jnp
from jax import lax
from jax.experimental import pallas as pl
from jax.experimental.pallas import tpu as pltpu
from jax.experimental.pallas import tpu_sc as plsc

N, E, D, H, C = 10000, 320000, 128, 128, 64

NC, NS = 2, 16
NW = NC * NS
E_PER_TILE = E // NW
CHUNK = 128
E_TILE_PAD = 10240
N_CHUNKS = E_TILE_PAD // CHUNK
N_ACC = 10240
N_DEG = 10240
ROWS_PER_SUB = N_ACC // NS

_sc_mesh = plsc.VectorSubcoreMesh(
    core_axis_name="c", subcore_axis_name="s", num_cores=NC, num_subcores=NS
)

_sc_params = pltpu.CompilerParams()
if "needs_layout_passes" in pltpu.CompilerParams.__dataclass_fields__:
    _sc_params = dataclasses.replace(_sc_params, needs_layout_passes=False)


@functools.partial(
    pl.kernel,
    out_type=[
        jax.ShapeDtypeStruct((NC, N_ACC, D), jnp.float32),
        jax.ShapeDtypeStruct((NW * N_DEG,), jnp.float32),
    ],
    mesh=_sc_mesh,
    scratch_types=[
        pltpu.VMEM((E_TILE_PAD // 2,), jnp.int32),
        pltpu.VMEM((CHUNK,), jnp.int32),
        pltpu.VMEM((CHUNK,), jnp.int32),
        pltpu.VMEM((CHUNK, D), jnp.float32),
        pltpu.VMEM((CHUNK, D), jnp.float32),
        pltpu.VMEM((N_DEG,), jnp.float32),
        pltpu.VMEM_SHARED((N_ACC, D), jnp.float32),
        pltpu.SemaphoreType.DMA,
        pltpu.SemaphoreType.DMA,
        pltpu.SemaphoreType.DMA,
        pltpu.SemaphoreType.DMA,
    ],
    compiler_params=_sc_params,
)
def _sc_aggregate(
    x_hbm, src_hbm, dst_hbm, zeros_d_hbm,
    agg_out, deg_out,
    src_v, dstb0, dstb1, rows0_v, rows1_v, deg_v, agg_sh,
    semr0, semr1, semi0, semi1,
):
    cid = lax.axis_index("c")
    sid = lax.axis_index("s")
    r0 = sid * ROWS_PER_SUB
    wid = cid * NS + sid
    ebase = wid * E_TILE_PAD

    pltpu.sync_copy(
        zeros_d_hbm.at[pl.ds(r0, ROWS_PER_SUB)],
        agg_sh.at[pl.ds(r0, ROWS_PER_SUB)],
    )

    @pl.loop(0, N_DEG // 16)
    def _(i):
        deg_v[pl.ds(i * 16, 16)] = jnp.zeros((16,), jnp.float32)

    plsc.subcore_barrier()

    one16 = jnp.ones((16,), jnp.float32)

    def scatter(rows_v, dstb):
        pltpu.sync_copy(rows_v, agg_sh.at[dstb], add=True)
        for k in range(CHUNK // 16):
            idx16 = dstb[pl.ds(k * 16, 16)]
            plsc.addupdate_scatter(deg_v, [idx16], one16)

    HALF = N_CHUNKS // 2

    for ph in range(2):
        pb = ebase + ph * HALF * CHUNK

        pltpu.sync_copy(src_hbm.at[pl.ds(pb, HALF * CHUNK)], src_v)

        def start_chunk(g, rows_v, dstb, semr, semi):
            idx = src_v.at[pl.ds(g * CHUNK, CHUNK)]
            pltpu.async_copy(x_hbm.at[idx], rows_v, semr)
            pltpu.async_copy(dst_hbm.at[pl.ds(pb + g * CHUNK, CHUNK)],
                             dstb, semi)

        def wait_chunk(g, rows_v, dstb, semr, semi):
            idx = src_v.at[pl.ds(g * CHUNK, CHUNK)]
            pltpu.make_async_copy(x_hbm.at[idx], rows_v, semr).wait()
            pltpu.make_async_copy(dst_hbm.at[pl.ds(pb + g * CHUNK, CHUNK)],
                                  dstb, semi).wait()

        start_chunk(0, rows0_v, dstb0, semr0, semi0)

        @pl.loop(0, HALF // 2 - 1)
        def _(p):
            i0 = 2 * p
            wait_chunk(i0, rows0_v, dstb0, semr0, semi0)
            start_chunk(i0 + 1, rows1_v, dstb1, semr1, semi1)
            scatter(rows0_v, dstb0)
            wait_chunk(i0 + 1, rows1_v, dstb1, semr1, semi1)
            start_chunk(i0 + 2, rows0_v, dstb0, semr0, semi0)
            scatter(rows1_v, dstb1)

        wait_chunk(HALF - 2, rows0_v, dstb0, semr0, semi0)
        start_chunk(HALF - 1, rows1_v, dstb1, semr1, semi1)
        scatter(rows0_v, dstb0)
        wait_chunk(HALF - 1, rows1_v, dstb1, semr1, semi1)
        scatter(rows1_v, dstb1)

    plsc.subcore_barrier()

    pltpu.sync_copy(
        agg_sh.at[pl.ds(r0, ROWS_PER_SUB)],
        agg_out.at[cid, pl.ds(r0, ROWS_PER_SUB)],
    )
    pltpu.sync_copy(deg_v, deg_out.at[pl.ds(wid * N_DEG, N_DEG)])


def _tc_body(
    agg_ref, deg_ref, x_ref, wl_ref, bl_ref, wr_ref, wlin_ref, blin_ref,
    out_ref, lab_ref,
):
    agg = agg_ref[0] + agg_ref[1]
    deg = jnp.sum(deg_ref[...], axis=1)
    agg = agg / jnp.maximum(deg, 1.0)[:, None]
    h = (
        jnp.dot(agg, wl_ref[...], preferred_element_type=jnp.float32)
        + bl_ref[...]
        + jnp.dot(x_ref[...], wr_ref[...], preferred_element_type=jnp.float32)
    )
    h = jnp.maximum(h, 0.0)
    logits = (
        jnp.dot(h, wlin_ref[...], preferred_element_type=jnp.float32)
        + blin_ref[...]
    )
    m = jnp.max(logits, axis=1, keepdims=True)
    lse = jnp.log(jnp.sum(jnp.exp(logits - m), axis=1, keepdims=True)) + m
    o = logits - lse
    out_ref[...] = o
    lab_ref[...] = jnp.argmax(o, axis=1).astype(jnp.int32)[:, None]


_TC_R = 1000


def _tc_head(agg_parts, deg_t, x, W_l, b_l2, W_r, W_lin, b_lin2):
    grid = (N // _TC_R,)
    return pl.pallas_call(
        _tc_body,
        grid=grid,
        in_specs=[
            pl.BlockSpec((NC, _TC_R, D), lambda i: (0, i, 0)),
            pl.BlockSpec((_TC_R, NW), lambda i: (i, 0)),
            pl.BlockSpec((_TC_R, D), lambda i: (i, 0)),
            pl.BlockSpec((D, H), lambda i: (0, 0)),
            pl.BlockSpec((1, H), lambda i: (0, 0)),
            pl.BlockSpec((D, H), lambda i: (0, 0)),
            pl.BlockSpec((H, C), lambda i: (0, 0)),
            pl.BlockSpec((1, C), lambda i: (0, 0)),
        ],
        out_specs=[
            pl.BlockSpec((_TC_R, C), lambda i: (i, 0)),
            pl.BlockSpec((_TC_R, 1), lambda i: (i, 0)),
        ],
        out_shape=[
            jax.ShapeDtypeStruct((N, C), jnp.float32),
            jax.ShapeDtypeStruct((N, 1), jnp.int32),
        ],
    )(agg_parts, deg_t, x, W_l, b_l2, W_r, W_lin, b_lin2)


def kernel(x, edge_index, W_l, b_l, W_r, W_lin, b_lin):
    pad = E_TILE_PAD - E_PER_TILE
    src = jnp.concatenate(
        [
            edge_index[0].reshape(NW, E_PER_TILE),
            jnp.zeros((NW, pad), jnp.int32),
        ],
        axis=1,
    ).reshape(-1)
    pad_rows = N + jnp.arange(pad, dtype=jnp.int32)[None, :]
    dst = jnp.concatenate(
        [
            edge_index[1].reshape(NW, E_PER_TILE),
            jnp.broadcast_to(pad_rows, (NW, pad)),
        ],
        axis=1,
    ).reshape(-1)
    zeros_d = jnp.zeros((N_ACC, D), jnp.float32)
    agg_parts, deg_flat = _sc_aggregate(x, src, dst, zeros_d)
    deg_t = deg_flat.reshape(NW, N_DEG).T[:N]
    out, lab2 = _tc_head(
        agg_parts, deg_t, x,
        W_l, b_l.reshape(1, H), W_r, W_lin, b_lin.reshape(1, C),
    )
    return lab2.reshape(N), out


# speedup vs baseline: 2.1969x; 2.1969x over previous
import dataclasses
import functools

import jax
import jax.numpy as jnp
from jax import lax
from jax.experimental import pallas as pl
from jax.experimental.pallas import tpu as pltpu
from jax.experimental.pallas import tpu_sc as plsc

N, E, D, H, C = 10000, 320000, 128, 128, 64

NC, NS = 2, 16
NW = NC * NS
E_PER_TILE = E // NW
CHUNK = 80
N_CHUNKS = E_PER_TILE // CHUNK
N_ACC = 10240
ROWS_PER_SUB = N_ACC // NS

_sc_mesh = plsc.VectorSubcoreMesh(
    core_axis_name="c", subcore_axis_name="s", num_cores=NC, num_subcores=NS
)

_sc_params = pltpu.CompilerParams()
if "needs_layout_passes" in pltpu.CompilerParams.__dataclass_fields__:
    _sc_params = dataclasses.replace(_sc_params, needs_layout_passes=False)


@functools.partial(
    pl.kernel,
    out_type=[
        jax.ShapeDtypeStruct((NC, N_ACC, D), jnp.float32),
        jax.ShapeDtypeStruct((NW * N,), jnp.float32),
    ],
    mesh=_sc_mesh,
    scratch_types=[
        pltpu.VMEM((E_PER_TILE,), jnp.int32),
        pltpu.VMEM((CHUNK,), jnp.int32),
        pltpu.VMEM((CHUNK,), jnp.int32),
        pltpu.VMEM((CHUNK, D), jnp.float32),
        pltpu.VMEM((CHUNK, D), jnp.float32),
        pltpu.VMEM((N,), jnp.float32),
        pltpu.VMEM_SHARED((N_ACC, D), jnp.float32),
        pltpu.SemaphoreType.DMA,
        pltpu.SemaphoreType.DMA,
        pltpu.SemaphoreType.DMA,
        pltpu.SemaphoreType.DMA,
    ],
    compiler_params=_sc_params,
)
def _sc_aggregate(
    x_hbm, src_hbm, dst_hbm, zeros_d_hbm,
    agg_out, deg_out,
    src_v, dstb0, dstb1, rows0_v, rows1_v, deg_v, agg_sh,
    semr0, semr1, semi0, semi1,
):
    cid = lax.axis_index("c")
    sid = lax.axis_index("s")
    r0 = sid * ROWS_PER_SUB
    wid = cid * NS + sid
    ebase = wid * E_PER_TILE

    pltpu.sync_copy(src_hbm.at[pl.ds(ebase, E_PER_TILE)], src_v)

    pltpu.sync_copy(
        zeros_d_hbm.at[pl.ds(r0, ROWS_PER_SUB)],
        agg_sh.at[pl.ds(r0, ROWS_PER_SUB)],
    )

    @pl.loop(0, N // 16)
    def _(i):
        deg_v[pl.ds(i * 16, 16)] = jnp.zeros((16,), jnp.float32)

    plsc.subcore_barrier()

    one16 = jnp.ones((16,), jnp.float32)

    def start_chunk(g, rows_v, dstb, semr, semi):
        idx = src_v.at[pl.ds(g * CHUNK, CHUNK)]
        pltpu.async_copy(x_hbm.at[idx], rows_v, semr)
        pltpu.async_copy(dst_hbm.at[pl.ds(ebase + g * CHUNK, CHUNK)], dstb, semi)

    def wait_chunk(g, rows_v, dstb, semr, semi):
        idx = src_v.at[pl.ds(g * CHUNK, CHUNK)]
        pltpu.make_async_copy(x_hbm.at[idx], rows_v, semr).wait()
        pltpu.make_async_copy(
            dst_hbm.at[pl.ds(ebase + g * CHUNK, CHUNK)], dstb, semi
        ).wait()

    def scatter(rows_v, dstb):
        pltpu.sync_copy(rows_v, agg_sh.at[dstb], add=True)
        for k in range(CHUNK // 16):
            idx16 = dstb[pl.ds(k * 16, 16)]
            plsc.addupdate_scatter(deg_v, [idx16], one16)

    start_chunk(0, rows0_v, dstb0, semr0, semi0)

    @pl.loop(0, (N_CHUNKS - 1) // 2)
    def _(p):
        i0 = 2 * p
        wait_chunk(i0, rows0_v, dstb0, semr0, semi0)
        start_chunk(i0 + 1, rows1_v, dstb1, semr1, semi1)
        scatter(rows0_v, dstb0)
        wait_chunk(i0 + 1, rows1_v, dstb1, semr1, semi1)
        start_chunk(i0 + 2, rows0_v, dstb0, semr0, semi0)
        scatter(rows1_v, dstb1)

    wait_chunk(N_CHUNKS - 1, rows0_v, dstb0, semr0, semi0)
    scatter(rows0_v, dstb0)

    plsc.subcore_barrier()

    pltpu.sync_copy(
        agg_sh.at[pl.ds(r0, ROWS_PER_SUB)],
        agg_out.at[cid, pl.ds(r0, ROWS_PER_SUB)],
    )
    pltpu.sync_copy(deg_v, deg_out.at[pl.ds(wid * N, N)])


def _tc_body(
    agg_ref, deg_ref, x_ref, wl_ref, bl_ref, wr_ref, wlin_ref, blin_ref,
    out_ref, lab_ref,
):
    agg = agg_ref[0] + agg_ref[1]
    deg = jnp.sum(deg_ref[...], axis=1)
    agg = agg / jnp.maximum(deg, 1.0)[:, None]
    h = (
        jnp.dot(agg, wl_ref[...], preferred_element_type=jnp.float32)
        + bl_ref[...]
        + jnp.dot(x_ref[...], wr_ref[...], preferred_element_type=jnp.float32)
    )
    h = jnp.maximum(h, 0.0)
    logits = (
        jnp.dot(h, wlin_ref[...], preferred_element_type=jnp.float32)
        + blin_ref[...]
    )
    m = jnp.max(logits, axis=1, keepdims=True)
    lse = jnp.log(jnp.sum(jnp.exp(logits - m), axis=1, keepdims=True)) + m
    o = logits - lse
    out_ref[...] = o
    lab_ref[...] = jnp.argmax(o, axis=1).astype(jnp.int32)[:, None]


_TC_R = 1000


def _tc_head(agg_parts, deg_t, x, W_l, b_l2, W_r, W_lin, b_lin2):
    grid = (N // _TC_R,)
    return pl.pallas_call(
        _tc_body,
        grid=grid,
        in_specs=[
            pl.BlockSpec((NC, _TC_R, D), lambda i: (0, i, 0)),
            pl.BlockSpec((_TC_R, NW), lambda i: (i, 0)),
            pl.BlockSpec((_TC_R, D), lambda i: (i, 0)),
            pl.BlockSpec((D, H), lambda i: (0, 0)),
            pl.BlockSpec((1, H), lambda i: (0, 0)),
            pl.BlockSpec((D, H), lambda i: (0, 0)),
            pl.BlockSpec((H, C), lambda i: (0, 0)),
            pl.BlockSpec((1, C), lambda i: (0, 0)),
        ],
        out_specs=[
            pl.BlockSpec((_TC_R, C), lambda i: (i, 0)),
            pl.BlockSpec((_TC_R, 1), lambda i: (i, 0)),
        ],
        out_shape=[
            jax.ShapeDtypeStruct((N, C), jnp.float32),
            jax.ShapeDtypeStruct((N, 1), jnp.int32),
        ],
    )(agg_parts, deg_t, x, W_l, b_l2, W_r, W_lin, b_lin2)


def kernel(x, edge_index, W_l, b_l, W_r, W_lin, b_lin):
    src = edge_index[0]
    dst = edge_index[1]
    zeros_d = jnp.zeros((N_ACC, D), jnp.float32)
    agg_parts, deg_flat = _sc_aggregate(x, src, dst, zeros_d)
    deg_t = deg_flat.reshape(NW, N).T
    out, lab2 = _tc_head(
        agg_parts, deg_t, x,
        W_l, b_l.reshape(1, H), W_r, W_lin, b_lin.reshape(1, C),
    )
    return lab2.reshape(N), out
